# baseline (speedup 1.0000x reference)
import jax
import jax.numpy as jnp
from jax import lax
from jax.experimental import pallas as pl
from jax.experimental.pallas import tpu as pltpu
from jax.experimental.pallas import tpu_sc as plsc

B = 4096
L = 200
D = 32
LANES = 16

NC = 2
NS = 16
NW = NC * NS
BPW = B // NW
CL = 8
N_CHUNKS = L // CL
TILE = 8 * 128


def _emb_body(table_hbm, xt_hbm, pe_hbm, scale_hbm, out_hbm,
              idx_v, rows_v, q_v, pe_flat_v, peb_v, scale_v, sem, sem_out):
    w = lax.axis_index("s") * NC + lax.axis_index("c")

    pltpu.sync_copy(pe_hbm, pe_flat_v)
    pltpu.sync_copy(scale_hbm, scale_v)
    sv = scale_v[...]
    iota = lax.iota(jnp.int32, LANES)

    def chunk_body(c, carry):
        l0 = c * CL
        pltpu.sync_copy(xt_hbm.at[pl.ds(l0, CL), pl.ds(w * BPW, BPW)], idx_v)
        copies = [
            pltpu.async_copy(
                table_hbm.at[idx_v.at[li]],
                rows_v.at[pl.ds(li * BPW, BPW)],
                sem,
            )
            for li in range(CL)
        ]

        @plsc.parallel_loop(0, CL * D, unroll=4)
        def peb_body(i):
            li = i // D
            d = i % D
            src = jnp.full((LANES,), (l0 + li) * D + d, jnp.int32)
            peb_v[i, pl.ds(0, LANES)] = plsc.load_gather(pe_flat_v, [src])

        for cp in copies:
            cp.wait()

        @plsc.parallel_loop(0, CL * D, unroll=2)
        def fuse_body(i):
            li = i // D
            d = i % D
            col = jnp.full((LANES,), d, jnp.int32)
            pe_d = peb_v[i, pl.ds(0, LANES)]
            qrow = li * 4 + d // 8
            qbase = (d % 8) * 128
            for blk in range(BPW // LANES):
                row_idx = iota + (li * BPW + blk * LANES)
                val = plsc.load_gather(rows_v, [row_idx, col])
                q_v[qrow, pl.ds(qbase + blk * LANES, LANES)] = (
                    val * sv + pe_d)

        pltpu.async_copy(
            q_v,
            out_hbm.at[pl.ds(l0 * 4, CL * 4), pl.ds(w * TILE, TILE)],
            sem_out,
        ).wait()
        return carry

    lax.fori_loop(0, N_CHUNKS, chunk_body, 0)


def kernel(x, table, pe, scale):
    xt = jnp.asarray(x, jnp.int32).T
    pe_flat = pe[:L].reshape(L * D)
    scale_v = jnp.broadcast_to(scale.astype(jnp.float32), (LANES,))
    mesh = plsc.VectorSubcoreMesh(core_axis_name="c", subcore_axis_name="s")
    q = pl.kernel(
        _emb_body,
        out_type=jax.ShapeDtypeStruct((L * (D // 8), NW * TILE), jnp.float32),
        mesh=mesh,
        compiler_params=pltpu.CompilerParams(
            use_tc_tiling_on_sc=False, needs_layout_passes=False),
        scratch_types=[
            pltpu.VMEM((CL, BPW), jnp.int32),
            pltpu.VMEM((CL * BPW, D), jnp.float32),
            pltpu.VMEM((CL * 4, TILE), jnp.float32),
            pltpu.VMEM((L * D,), jnp.float32),
            pltpu.VMEM((CL * D, LANES), jnp.float32),
            pltpu.VMEM((LANES,), jnp.float32),
            pltpu.SemaphoreType.DMA,
            pltpu.SemaphoreType.DMA,
        ],
    )(table, xt, pe_flat, scale_v)
    out = (
        q.reshape(L, D // 8, NW, 8, BPW)
        .transpose(2, 4, 0, 1, 3)
        .reshape(B, L, D)
    )
    return out


# speedup vs baseline: 1.2051x; 1.2051x over previous
import jax
import jax.numpy as jnp
from jax import lax
from jax.experimental import pallas as pl
from jax.experimental.pallas import tpu as pltpu
from jax.experimental.pallas import tpu_sc as plsc

B = 4096
L = 200
D = 32
LANES = 16

NC = 2
NS = 16
NW = NC * NS
BPW = B // NW
CL = 8
N_CHUNKS = L // CL
TILE = 8 * 128
RP = D + 1


def _emb_body(table_hbm, xt_hbm, pe_hbm, scale_hbm, out_hbm,
              idx_v, rows_v, rows2_v, q_v, pe_v, scale_v, sem, sem_out):
    w = lax.axis_index("s") * NC + lax.axis_index("c")

    pltpu.sync_copy(pe_hbm.at[pl.ds(0, L)], pe_v)
    pltpu.sync_copy(scale_hbm, scale_v)
    sv = scale_v[...]
    iota = lax.iota(jnp.int32, LANES)

    def chunk_body(c, carry):
        l0 = c * CL
        pltpu.sync_copy(xt_hbm.at[pl.ds(l0, CL), pl.ds(w * BPW, BPW)], idx_v)
        copies = [
            pltpu.async_copy(
                table_hbm.at[idx_v.at[li]],
                rows_v.at[pl.ds(li * BPW, BPW)],
                sem,
            )
            for li in range(CL)
        ]
        for cp in copies:
            cp.wait()

        for li in range(CL):
            l = l0 + li
            pe_lo = pe_v[l, pl.ds(0, LANES)]
            pe_hi = pe_v[l, pl.ds(LANES, LANES)]

            @plsc.parallel_loop(0, BPW // 8, unroll=2)
            def bl_body(bl8, li=li, pe_lo=pe_lo, pe_hi=pe_hi):
                for s in range(8):
                    r = li * BPW + bl8 * 8 + s
                    rows2_v[r, pl.ds(0, LANES)] = (
                        rows_v[r, pl.ds(0, LANES)] * sv + pe_lo)
                    rows2_v[r, pl.ds(LANES, LANES)] = (
                        rows_v[r, pl.ds(LANES, LANES)] * sv + pe_hi)

        @plsc.parallel_loop(0, CL * (BPW // LANES), unroll=2)
        def blk_body(i):
            li = i // (BPW // LANES)
            blk = i % (BPW // LANES)
            row_idx = iota + (li * BPW + blk * LANES)
            qcol = blk * LANES
            for d in range(D):
                col = jnp.full((LANES,), d, jnp.int32)
                val = plsc.load_gather(rows2_v, [row_idx, col])
                q_v[li * 4 + d // 8,
                    pl.ds((d % 8) * 128 + qcol, LANES)] = val

        pltpu.async_copy(
            q_v,
            out_hbm.at[pl.ds(l0 * 4, CL * 4), pl.ds(w * TILE, TILE)],
            sem_out,
        ).wait()
        return carry

    lax.fori_loop(0, N_CHUNKS, chunk_body, 0)


def kernel(x, table, pe, scale):
    xt = jnp.asarray(x, jnp.int32).T
    scale_v = jnp.broadcast_to(scale.astype(jnp.float32), (LANES,))
    mesh = plsc.VectorSubcoreMesh(core_axis_name="c", subcore_axis_name="s")
    q = pl.kernel(
        _emb_body,
        out_type=jax.ShapeDtypeStruct((L * (D // 8), NW * TILE), jnp.float32),
        mesh=mesh,
        compiler_params=pltpu.CompilerParams(
            use_tc_tiling_on_sc=False, needs_layout_passes=False),
        scratch_types=[
            pltpu.VMEM((CL, BPW), jnp.int32),
            pltpu.VMEM((CL * BPW, D), jnp.float32),
            pltpu.VMEM((CL * BPW, RP), jnp.float32),
            pltpu.VMEM((CL * 4, TILE), jnp.float32),
            pltpu.VMEM((L, D), jnp.float32),
            pltpu.VMEM((LANES,), jnp.float32),
            pltpu.SemaphoreType.DMA,
            pltpu.SemaphoreType.DMA,
        ],
    )(table, xt, pe, scale_v)
    out = (
        q.reshape(L, D // 8, NW, 8, BPW)
        .transpose(2, 4, 0, 1, 3)
        .reshape(B, L, D)
    )
    return out


# speedup vs baseline: 1.4420x; 1.1966x over previous
import jax
import jax.numpy as jnp
from jax import lax
from jax.experimental import pallas as pl
from jax.experimental.pallas import tpu as pltpu
from jax.experimental.pallas import tpu_sc as plsc

B = 4096
L = 200
D = 32
LANES = 16

NC = 2
NS = 16
NW = NC * NS
BPW = B // NW
CL = 8
N_CHUNKS = L // CL
TILE = 8 * 128
QP = 129


def _emb_body(table_hbm, xt_hbm, pe_hbm, scale_hbm, out_hbm,
              idx_v, rows_v, q_v, pe_v, scale_v, sem, sem_out):
    w = lax.axis_index("s") * NC + lax.axis_index("c")

    pltpu.sync_copy(pe_hbm.at[pl.ds(0, L)], pe_v)
    pltpu.sync_copy(scale_hbm, scale_v)
    sv = scale_v[...]
    iota = lax.iota(jnp.int32, LANES)
    di_v = iota & 7

    def chunk_body(c, carry):
        l0 = c * CL
        pltpu.sync_copy(xt_hbm.at[pl.ds(l0, CL), pl.ds(w * BPW, BPW)], idx_v)
        copies = [
            pltpu.async_copy(
                table_hbm.at[idx_v.at[li]],
                rows_v.at[pl.ds(li * BPW, BPW)],
                sem,
            )
            for li in range(CL)
        ]
        for cp in copies:
            cp.wait()

        for li in range(CL):
            l = l0 + li
            pe_lo = pe_v[l, pl.ds(0, LANES)]
            pe_hi = pe_v[l, pl.ds(LANES, LANES)]
            k_lo = (iota >> 3) + li * 4
            k_hi = k_lo + 2

            @plsc.parallel_loop(0, BPW, unroll=2)
            def bl_body(bl, li=li, pe_lo=pe_lo, pe_hi=pe_hi,
                        k_lo=k_lo, k_hi=k_hi):
                r = li * BPW + bl
                col = jnp.full((LANES,), bl, jnp.int32)
                v_lo = rows_v[r, pl.ds(0, LANES)] * sv + pe_lo
                v_hi = rows_v[r, pl.ds(LANES, LANES)] * sv + pe_hi
                plsc.store_scatter(q_v, [k_lo, di_v, col], v_lo)
                plsc.store_scatter(q_v, [k_hi, di_v, col], v_hi)

        pltpu.async_copy(
            q_v.at[:, :, pl.ds(0, BPW)],
            out_hbm.at[pl.ds(l0 * 4, CL * 4), pl.ds(w * 8, 8), :],
            sem_out,
        ).wait()
        return carry

    lax.fori_loop(0, N_CHUNKS, chunk_body, 0)


def kernel(x, table, pe, scale):
    xt = jnp.asarray(x, jnp.int32).T
    scale_v = jnp.broadcast_to(scale.astype(jnp.float32), (LANES,))
    mesh = plsc.VectorSubcoreMesh(core_axis_name="c", subcore_axis_name="s")
    q = pl.kernel(
        _emb_body,
        out_type=jax.ShapeDtypeStruct((L * (D // 8), NW * 8, BPW),
                                      jnp.float32),
        mesh=mesh,
        compiler_params=pltpu.CompilerParams(
            use_tc_tiling_on_sc=False, needs_layout_passes=False),
        scratch_types=[
            pltpu.VMEM((CL, BPW), jnp.int32),
            pltpu.VMEM((CL * BPW, D), jnp.float32),
            pltpu.VMEM((CL * 4, 8, QP), jnp.float32),
            pltpu.VMEM((L, D), jnp.float32),
            pltpu.VMEM((LANES,), jnp.float32),
            pltpu.SemaphoreType.DMA,
            pltpu.SemaphoreType.DMA,
        ],
    )(table, xt, pe, scale_v)
    out = (
        q.reshape(L, D // 8, NW, 8, BPW)
        .transpose(2, 4, 0, 1, 3)
        .reshape(B, L, D)
    )
    return out


# speedup vs baseline: 1.4740x; 1.0222x over previous
import jax
import jax.numpy as jnp
from jax import lax
from jax.experimental import pallas as pl
from jax.experimental.pallas import tpu as pltpu
from jax.experimental.pallas import tpu_sc as plsc

B = 4096
L = 200
D = 32
LANES = 16

NC = 2
NS = 16
NW = NC * NS
BPW = B // NW
CL = 8
N_CHUNKS = L // CL
TILE = 8 * 128
QP = 129


def _emb_body(table_hbm, xt_hbm, pe_hbm, scale_hbm, out_hbm,
              idx_v, rows_v, q_v, pe_v, scale_v, sem, sem_out):
    w = lax.axis_index("s") * NC + lax.axis_index("c")

    pltpu.sync_copy(pe_hbm.at[pl.ds(0, L)], pe_v)
    pltpu.sync_copy(scale_hbm, scale_v)
    pltpu.sync_copy(xt_hbm.at[:, pl.ds(w * BPW, BPW)], idx_v)
    sv = scale_v[...]
    iota = lax.iota(jnp.int32, LANES)
    di_v = iota & 7

    def chunk_body(c, carry):
        l0 = c * CL
        copies = [
            pltpu.async_copy(
                table_hbm.at[idx_v.at[l0 + li]],
                rows_v.at[pl.ds(li * BPW, BPW)],
                sem,
            )
            for li in range(CL)
        ]
        for cp in copies:
            cp.wait()

        for li in range(CL):
            l = l0 + li
            pe_lo = pe_v[l, pl.ds(0, LANES)]
            pe_hi = pe_v[l, pl.ds(LANES, LANES)]
            k_lo = (iota >> 3) + li * 4
            k_hi = k_lo + 2

            @plsc.parallel_loop(0, BPW, unroll=2)
            def bl_body(bl, li=li, pe_lo=pe_lo, pe_hi=pe_hi,
                        k_lo=k_lo, k_hi=k_hi):
                r = li * BPW + bl
                col = jnp.full((LANES,), bl, jnp.int32)
                v_lo = rows_v[r, pl.ds(0, LANES)] * sv + pe_lo
                v_hi = rows_v[r, pl.ds(LANES, LANES)] * sv + pe_hi
                plsc.store_scatter(q_v, [k_lo, di_v, col], v_lo)
                plsc.store_scatter(q_v, [k_hi, di_v, col], v_hi)

        pltpu.async_copy(
            q_v.at[:, :, pl.ds(0, BPW)],
            out_hbm.at[pl.ds(l0 * 4, CL * 4), pl.ds(w * 8, 8), :],
            sem_out,
        ).wait()
        return carry

    lax.fori_loop(0, N_CHUNKS, chunk_body, 0)


def kernel(x, table, pe, scale):
    xt = jnp.asarray(x, jnp.int32).T
    scale_v = jnp.broadcast_to(scale.astype(jnp.float32), (LANES,))
    mesh = plsc.VectorSubcoreMesh(core_axis_name="c", subcore_axis_name="s")
    q = pl.kernel(
        _emb_body,
        out_type=jax.ShapeDtypeStruct((L * (D // 8), NW * 8, BPW),
                                      jnp.float32),
        mesh=mesh,
        compiler_params=pltpu.CompilerParams(
            use_tc_tiling_on_sc=False, needs_layout_passes=False),
        scratch_types=[
            pltpu.VMEM((L, BPW), jnp.int32),
            pltpu.VMEM((CL * BPW, D), jnp.float32),
            pltpu.VMEM((CL * 4, 8, QP), jnp.float32),
            pltpu.VMEM((L, D), jnp.float32),
            pltpu.VMEM((LANES,), jnp.float32),
            pltpu.SemaphoreType.DMA,
            pltpu.SemaphoreType.DMA,
        ],
    )(table, xt, pe, scale_v)
    out = (
        q.reshape(L, D // 8, NW, 8, BPW)
        .transpose(2, 4, 0, 1, 3)
        .reshape(B, L, D)
    )
    return out


# speedup vs baseline: 1.4970x; 1.0156x over previous
import jax
import jax.numpy as jnp
from jax import lax
from jax.experimental import pallas as pl
from jax.experimental.pallas import tpu as pltpu
from jax.experimental.pallas import tpu_sc as plsc

B = 4096
L = 200
D = 32
LANES = 16

NC = 2
NS = 16
NW = NC * NS
BPW = B // NW
CL = 4
N_PAIRS = L // (2 * CL)
TILE = 8 * 128
QP = 129


def _emb_body(table_hbm, xt_hbm, pe_hbm, scale_hbm, out_hbm,
              idx_v, rows_a, rows_b, q_a, q_b, pe_v, scale_v,
              sem_a, sem_b, sem_out):
    w = lax.axis_index("s") * NC + lax.axis_index("c")

    pltpu.sync_copy(pe_hbm.at[pl.ds(0, L)], pe_v)
    pltpu.sync_copy(scale_hbm, scale_v)
    pltpu.sync_copy(xt_hbm.at[:, pl.ds(w * BPW, BPW)], idx_v)
    sv = scale_v[...]
    iota = lax.iota(jnp.int32, LANES)
    di_v = iota & 7

    def compute(rows_v, q_v, l0):
        for li in range(CL):
            l = l0 + li
            pe_lo = pe_v[l, pl.ds(0, LANES)]
            pe_hi = pe_v[l, pl.ds(LANES, LANES)]
            k_lo = (iota >> 3) + li * 4
            k_hi = k_lo + 2

            @plsc.parallel_loop(0, BPW, unroll=2)
            def bl_body(bl, li=li, pe_lo=pe_lo, pe_hi=pe_hi,
                        k_lo=k_lo, k_hi=k_hi):
                r = li * BPW + bl
                col = jnp.full((LANES,), bl, jnp.int32)
                v_lo = rows_v[r, pl.ds(0, LANES)] * sv + pe_lo
                v_hi = rows_v[r, pl.ds(LANES, LANES)] * sv + pe_hi
                plsc.store_scatter(q_v, [k_lo, di_v, col], v_lo)
                plsc.store_scatter(q_v, [k_hi, di_v, col], v_hi)

    def gathers(rows_v, l0, sem):
        return [
            pltpu.async_copy(
                table_hbm.at[idx_v.at[l0 + li]],
                rows_v.at[pl.ds(li * BPW, BPW)],
                sem,
            )
            for li in range(CL)
        ]

    def writeback(q_v, l0):
        return pltpu.async_copy(
            q_v.at[:, :, pl.ds(0, BPW)],
            out_hbm.at[pl.ds(l0 * 4, CL * 4), pl.ds(w * 8, 8), :],
            sem_out,
        )

    def pair_body(p, carry):
        l0a = p * (2 * CL)
        l0b = l0a + CL
        gs_a = gathers(rows_a, l0a, sem_a)
        gs_b = gathers(rows_b, l0b, sem_b)
        for cp in gs_a:
            cp.wait()
        compute(rows_a, q_a, l0a)
        wb_a = writeback(q_a, l0a)
        for cp in gs_b:
            cp.wait()
        compute(rows_b, q_b, l0b)
        wb_b = writeback(q_b, l0b)
        wb_a.wait()
        wb_b.wait()
        return carry

    lax.fori_loop(0, N_PAIRS, pair_body, 0)


def kernel(x, table, pe, scale):
    xt = jnp.asarray(x, jnp.int32).T
    scale_v = jnp.broadcast_to(scale.astype(jnp.float32), (LANES,))
    mesh = plsc.VectorSubcoreMesh(core_axis_name="c", subcore_axis_name="s")
    q = pl.kernel(
        _emb_body,
        out_type=jax.ShapeDtypeStruct((L * (D // 8), NW * 8, BPW),
                                      jnp.float32),
        mesh=mesh,
        compiler_params=pltpu.CompilerParams(
            use_tc_tiling_on_sc=False, needs_layout_passes=False),
        scratch_types=[
            pltpu.VMEM((L, BPW), jnp.int32),
            pltpu.VMEM((CL * BPW, D), jnp.float32),
            pltpu.VMEM((CL * BPW, D), jnp.float32),
            pltpu.VMEM((CL * 4, 8, QP), jnp.float32),
            pltpu.VMEM((CL * 4, 8, QP), jnp.float32),
            pltpu.VMEM((L, D), jnp.float32),
            pltpu.VMEM((LANES,), jnp.float32),
            pltpu.SemaphoreType.DMA,
            pltpu.SemaphoreType.DMA,
            pltpu.SemaphoreType.DMA,
        ],
    )(table, xt, pe, scale_v)
    out = (
        q.reshape(L, D // 8, NW, 8, BPW)
        .transpose(2, 4, 0, 1, 3)
        .reshape(B, L, D)
    )
    return out
